# Initial kernel scaffold; baseline (speedup 1.0000x reference)
import jax
import jax.numpy as jnp
from jax.experimental import pallas as pl


def kernel(idx, emb):
    raise NotImplementedError("write your pallas kernel here")


# speedup vs baseline: 2.4875x; 2.4875x over previous
import functools

import jax
import jax.numpy as jnp
from jax import lax
from jax.experimental import pallas as pl
from jax.experimental.pallas import tpu as pltpu
from jax.experimental.pallas import tpu_sc as plsc

NUM_CLS = 1000000
EMB_DIM = 16
BATCH = 16384
HIST = 200

B = BATCH * HIST
NC = 2
NS = 16
NW = NC * NS
BPW = B // NW
IDXW = 128
CH = 2048
K = CH // IDXW
NCHUNK = BPW // CH

_mesh = plsc.VectorSubcoreMesh(core_axis_name="c", subcore_axis_name="s")


@functools.partial(
    pl.kernel,
    out_type=jax.ShapeDtypeStruct((B, EMB_DIM), jnp.float32),
    mesh=_mesh,
    scratch_types=[
        pltpu.VMEM((K, IDXW), jnp.int32),
        pltpu.VMEM((CH, EMB_DIM), jnp.float32),
        pltpu.SemaphoreType.DMA,
    ],
    compiler_params=pltpu.CompilerParams(use_tc_tiling_on_sc=False),
)
def _sc_gather(idx_hbm, emb_hbm, out_hbm, idx_v, rows_v, sem):
    wid = lax.axis_index("s") * NC + lax.axis_index("c")
    base = wid * BPW

    def chunk(g, carry):
        off = pl.multiple_of(base + g * CH, CH)
        row = pl.multiple_of(wid * (BPW // IDXW) + g * K, K)
        pltpu.sync_copy(idx_hbm.at[pl.ds(row, K)], idx_v)
        copies = [
            pltpu.async_copy(
                emb_hbm.at[idx_v.at[j]],
                rows_v.at[pl.ds(j * IDXW, IDXW)],
                sem,
            )
            for j in range(K)
        ]
        for c in copies:
            c.wait()
        pltpu.sync_copy(rows_v, out_hbm.at[pl.ds(off, CH)])
        return carry

    lax.fori_loop(0, NCHUNK, chunk, 0)


def kernel(idx, emb):
    idx2d = idx.reshape(B // IDXW, IDXW).astype(jnp.int32)
    out = _sc_gather(idx2d, emb)
    return out.reshape(BATCH, HIST, EMB_DIM)


# speedup vs baseline: 2.5294x; 1.0169x over previous
import functools

import jax
import jax.numpy as jnp
from jax import lax
from jax.experimental import pallas as pl
from jax.experimental.pallas import tpu as pltpu
from jax.experimental.pallas import tpu_sc as plsc

NUM_CLS = 1000000
EMB_DIM = 16
BATCH = 16384
HIST = 200

B = BATCH * HIST
NC = 2
NS = 16
NW = NC * NS
BPW = B // NW
IDXW = 128
CH = 2048
K = CH // IDXW
NCHUNK = BPW // CH
NBUF = 2

_mesh = plsc.VectorSubcoreMesh(core_axis_name="c", subcore_axis_name="s")


@functools.partial(
    pl.kernel,
    out_type=jax.ShapeDtypeStruct((B, EMB_DIM), jnp.float32),
    mesh=_mesh,
    scratch_types=[
        pltpu.VMEM((K, IDXW), jnp.int32),
        pltpu.VMEM((K, IDXW), jnp.int32),
        pltpu.VMEM((CH, EMB_DIM), jnp.float32),
        pltpu.VMEM((CH, EMB_DIM), jnp.float32),
        pltpu.SemaphoreType.DMA,
        pltpu.SemaphoreType.DMA,
        pltpu.SemaphoreType.DMA,
        pltpu.SemaphoreType.DMA,
        pltpu.SemaphoreType.DMA,
        pltpu.SemaphoreType.DMA,
    ],
    compiler_params=pltpu.CompilerParams(use_tc_tiling_on_sc=False),
)
def _sc_gather(idx_hbm, emb_hbm, out_hbm,
               idx_v0, idx_v1, rows_v0, rows_v1,
               s_i0, s_i1, s_g0, s_g1, s_o0, s_o1):
    wid = lax.axis_index("s") * NC + lax.axis_index("c")
    base = wid * BPW
    base_row = wid * (BPW // IDXW)
    idx_bufs = (idx_v0, idx_v1)
    row_bufs = (rows_v0, rows_v1)
    s_i = (s_i0, s_i1)
    s_g = (s_g0, s_g1)
    s_o = (s_o0, s_o1)

    def start_idx(g, b):
        row = pl.multiple_of(base_row + g * K, K)
        pltpu.async_copy(idx_hbm.at[pl.ds(row, K)], idx_bufs[b], s_i[b])

    def wait_idx(b):
        pltpu.make_async_copy(idx_hbm.at[pl.ds(0, K)], idx_bufs[b], s_i[b]).wait()

    def wait_out(b):
        pltpu.make_async_copy(
            row_bufs[b], out_hbm.at[pl.ds(0, CH)], s_o[b]).wait()

    start_idx(0, 0)
    start_idx(1, 1)

    def outer(t, carry):
        for b in range(NBUF):
            g = t * NBUF + b
            wait_idx(b)

            @pl.when(t > 0)
            def _():
                wait_out(b)

            copies = [
                pltpu.async_copy(
                    emb_hbm.at[idx_bufs[b].at[j]],
                    row_bufs[b].at[pl.ds(j * IDXW, IDXW)],
                    s_g[b],
                )
                for j in range(K)
            ]
            for c in copies:
                c.wait()

            off = pl.multiple_of(base + g * CH, CH)
            pltpu.async_copy(row_bufs[b], out_hbm.at[pl.ds(off, CH)], s_o[b])

            @pl.when(g + NBUF < NCHUNK)
            def _():
                start_idx(g + NBUF, b)
        return carry

    lax.fori_loop(0, NCHUNK // NBUF, outer, 0)
    wait_out(0)
    wait_out(1)


def kernel(idx, emb):
    idx2d = idx.reshape(B // IDXW, IDXW).astype(jnp.int32)
    out = _sc_gather(idx2d, emb)
    return out.reshape(BATCH, HIST, EMB_DIM)


# speedup vs baseline: 2.5317x; 1.0009x over previous
import functools

import jax
import jax.numpy as jnp
from jax import lax
from jax.experimental import pallas as pl
from jax.experimental.pallas import tpu as pltpu
from jax.experimental.pallas import tpu_sc as plsc

NUM_CLS = 1000000
EMB_DIM = 16
BATCH = 16384
HIST = 200

B = BATCH * HIST
NC = 2
NS = 16
NW = NC * NS
BPW = B // NW
IDXW = 128
CH = 2048
K = CH // IDXW
NCHUNK = BPW // CH
NBUF = 2

_mesh = plsc.VectorSubcoreMesh(core_axis_name="c", subcore_axis_name="s")


@functools.partial(
    pl.kernel,
    out_type=jax.ShapeDtypeStruct((B, EMB_DIM), jnp.float32),
    mesh=_mesh,
    scratch_types=[
        pltpu.VMEM((CH,), jnp.int32),
        pltpu.VMEM((CH,), jnp.int32),
        pltpu.VMEM((CH, EMB_DIM), jnp.float32),
        pltpu.VMEM((CH, EMB_DIM), jnp.float32),
        pltpu.SemaphoreType.DMA,
        pltpu.SemaphoreType.DMA,
        pltpu.SemaphoreType.DMA,
        pltpu.SemaphoreType.DMA,
        pltpu.SemaphoreType.DMA,
        pltpu.SemaphoreType.DMA,
    ],
    compiler_params=pltpu.CompilerParams(use_tc_tiling_on_sc=False),
)
def _sc_gather(idx_hbm, emb_hbm, out_hbm,
               idx_v0, idx_v1, rows_v0, rows_v1,
               s_i0, s_i1, s_g0, s_g1, s_o0, s_o1):
    wid = lax.axis_index("s") * NC + lax.axis_index("c")
    base = wid * BPW
    base_row = wid * (BPW // IDXW)
    idx_bufs = (idx_v0, idx_v1)
    row_bufs = (rows_v0, rows_v1)
    s_i = (s_i0, s_i1)
    s_g = (s_g0, s_g1)
    s_o = (s_o0, s_o1)

    def start_idx(g, b):
        off = pl.multiple_of(base + g * CH, CH)
        pltpu.async_copy(idx_hbm.at[pl.ds(off, CH)], idx_bufs[b], s_i[b])

    def wait_idx(b):
        pltpu.make_async_copy(idx_hbm.at[pl.ds(0, CH)], idx_bufs[b], s_i[b]).wait()

    def wait_out(b):
        pltpu.make_async_copy(
            row_bufs[b], out_hbm.at[pl.ds(0, CH)], s_o[b]).wait()

    start_idx(0, 0)
    start_idx(1, 1)

    def outer(t, carry):
        for b in range(NBUF):
            g = t * NBUF + b
            wait_idx(b)

            @pl.when(t > 0)
            def _():
                wait_out(b)

            pltpu.async_copy(
                emb_hbm.at[idx_bufs[b]], row_bufs[b], s_g[b]).wait()

            off = pl.multiple_of(base + g * CH, CH)
            pltpu.async_copy(row_bufs[b], out_hbm.at[pl.ds(off, CH)], s_o[b])

            @pl.when(g + NBUF < NCHUNK)
            def _():
                start_idx(g + NBUF, b)
        return carry

    lax.fori_loop(0, NCHUNK // NBUF, outer, 0)
    wait_out(0)
    wait_out(1)


def kernel(idx, emb):
    idx_flat = idx.reshape(B).astype(jnp.int32)
    out = _sc_gather(idx_flat, emb)
    return out.reshape(BATCH, HIST, EMB_DIM)


# speedup vs baseline: 2.5436x; 1.0047x over previous
import functools

import jax
import jax.numpy as jnp
from jax import lax
from jax.experimental import pallas as pl
from jax.experimental.pallas import tpu as pltpu
from jax.experimental.pallas import tpu_sc as plsc

NUM_CLS = 1000000
EMB_DIM = 16
BATCH = 16384
HIST = 200

NC = 2
NS = 16
NW = NC * NS
RPW = BATCH // NW
CHR = 16
NCHUNK = RPW // CHR
NBUF = 2

_mesh = plsc.VectorSubcoreMesh(core_axis_name="c", subcore_axis_name="s")


@functools.partial(
    pl.kernel,
    out_type=jax.ShapeDtypeStruct((BATCH, HIST, EMB_DIM), jnp.float32),
    mesh=_mesh,
    scratch_types=[
        pltpu.VMEM((CHR, HIST), jnp.int32),
        pltpu.VMEM((CHR, HIST), jnp.int32),
        pltpu.VMEM((CHR, HIST, EMB_DIM), jnp.float32),
        pltpu.VMEM((CHR, HIST, EMB_DIM), jnp.float32),
        pltpu.SemaphoreType.DMA,
        pltpu.SemaphoreType.DMA,
        pltpu.SemaphoreType.DMA,
        pltpu.SemaphoreType.DMA,
        pltpu.SemaphoreType.DMA,
        pltpu.SemaphoreType.DMA,
    ],
    compiler_params=pltpu.CompilerParams(use_tc_tiling_on_sc=False),
)
def _sc_gather(idx_hbm, emb_hbm, out_hbm,
               idx_v0, idx_v1, rows_v0, rows_v1,
               s_i0, s_i1, s_g0, s_g1, s_o0, s_o1):
    wid = lax.axis_index("s") * NC + lax.axis_index("c")
    base = wid * RPW
    idx_bufs = (idx_v0, idx_v1)
    row_bufs = (rows_v0, rows_v1)
    s_i = (s_i0, s_i1)
    s_g = (s_g0, s_g1)
    s_o = (s_o0, s_o1)

    def start_idx(g, b):
        r0 = pl.multiple_of(base + g * CHR, CHR)
        pltpu.async_copy(idx_hbm.at[pl.ds(r0, CHR)], idx_bufs[b], s_i[b])

    def wait_idx(b):
        pltpu.make_async_copy(
            idx_hbm.at[pl.ds(0, CHR)], idx_bufs[b], s_i[b]).wait()

    def wait_out(b):
        pltpu.make_async_copy(
            row_bufs[b], out_hbm.at[pl.ds(0, CHR)], s_o[b]).wait()

    start_idx(0, 0)
    start_idx(1, 1)

    def outer(t, carry):
        for b in range(NBUF):
            g = t * NBUF + b
            wait_idx(b)

            @pl.when(t > 0)
            def _():
                wait_out(b)

            copies = [
                pltpu.async_copy(
                    emb_hbm.at[idx_bufs[b].at[j]],
                    row_bufs[b].at[j],
                    s_g[b],
                )
                for j in range(CHR)
            ]
            for c in copies:
                c.wait()

            r0 = pl.multiple_of(base + g * CHR, CHR)
            pltpu.async_copy(row_bufs[b], out_hbm.at[pl.ds(r0, CHR)], s_o[b])

            @pl.when(g + NBUF < NCHUNK)
            def _():
                start_idx(g + NBUF, b)
        return carry

    lax.fori_loop(0, NCHUNK // NBUF, outer, 0)
    wait_out(0)
    wait_out(1)


def kernel(idx, emb):
    return _sc_gather(idx.astype(jnp.int32), emb)


# speedup vs baseline: 3.6726x; 1.4439x over previous
import functools

import jax
import jax.numpy as jnp
from jax import lax
from jax.experimental import pallas as pl
from jax.experimental.pallas import tpu as pltpu
from jax.experimental.pallas import tpu_sc as plsc

NUM_CLS = 1000000
EMB_DIM = 16
BATCH = 16384
HIST = 200

NC = 2
NS = 16
NW = NC * NS
BW = BATCH // NW
HCH = 2
NCHUNK = HIST // HCH
NBUF = 2
LANES = 16

_mesh = plsc.VectorSubcoreMesh(core_axis_name="c", subcore_axis_name="s")


@functools.partial(
    pl.kernel,
    out_type=jax.ShapeDtypeStruct((HIST, EMB_DIM, BATCH), jnp.float32),
    mesh=_mesh,
    scratch_types=[
        pltpu.VMEM((HCH, BW), jnp.int32),
        pltpu.VMEM((HCH, BW), jnp.int32),
        pltpu.VMEM((HCH * BW, EMB_DIM), jnp.float32),
        pltpu.VMEM((HCH * BW, EMB_DIM), jnp.float32),
        pltpu.VMEM((HCH, EMB_DIM, BW), jnp.float32),
        pltpu.VMEM((HCH, EMB_DIM, BW), jnp.float32),
        pltpu.SemaphoreType.DMA,
        pltpu.SemaphoreType.DMA,
        pltpu.SemaphoreType.DMA,
        pltpu.SemaphoreType.DMA,
        pltpu.SemaphoreType.DMA,
        pltpu.SemaphoreType.DMA,
    ],
    compiler_params=pltpu.CompilerParams(
        use_tc_tiling_on_sc=False, needs_layout_passes=False),
)
def _sc_gather(idx_hbm, emb_hbm, out_hbm,
               idx_v0, idx_v1, rows_v0, rows_v1, trows_v0, trows_v1,
               s_i0, s_i1, s_g0, s_g1, s_o0, s_o1):
    wid = lax.axis_index("s") * NC + lax.axis_index("c")
    b0 = pl.multiple_of(wid * BW, BW)
    idx_bufs = (idx_v0, idx_v1)
    row_bufs = (rows_v0, rows_v1)
    trow_bufs = (trows_v0, trows_v1)
    s_i = (s_i0, s_i1)
    s_g = (s_g0, s_g1)
    s_o = (s_o0, s_o1)

    lane_iota = lax.iota(jnp.int32, LANES)

    def start_idx(g, b):
        h0 = pl.multiple_of(g * HCH, HCH)
        pltpu.async_copy(
            idx_hbm.at[pl.ds(h0, HCH), pl.ds(b0, BW)], idx_bufs[b], s_i[b])

    def wait_idx(b):
        pltpu.make_async_copy(
            idx_hbm.at[pl.ds(0, HCH), pl.ds(0, BW)], idx_bufs[b], s_i[b]).wait()

    def wait_out(b):
        pltpu.make_async_copy(
            trow_bufs[b],
            out_hbm.at[pl.ds(0, HCH), pl.ds(0, EMB_DIM), pl.ds(0, BW)],
            s_o[b]).wait()

    start_idx(0, 0)
    start_idx(1, 1)

    def outer(t, carry):
        for b in range(NBUF):
            g = t * NBUF + b
            wait_idx(b)

            @pl.when(t > 0)
            def _():
                wait_out(b)

            copies = [
                pltpu.async_copy(
                    emb_hbm.at[idx_bufs[b].at[hh]],
                    row_bufs[b].at[pl.ds(hh * BW, BW)],
                    s_g[b],
                )
                for hh in range(HCH)
            ]
            for c in copies:
                c.wait()

            def transpose_grp(i, carry2):
                base = pl.multiple_of(i * LANES, LANES)
                row_idx = base + lane_iota
                for hh in range(HCH):
                    for e in range(EMB_DIM):
                        vec = plsc.load_gather(
                            row_bufs[b],
                            [hh * BW + row_idx,
                             jnp.full((LANES,), e, jnp.int32)],
                        )
                        trow_bufs[b][hh, e, pl.ds(base, LANES)] = vec
                return carry2

            lax.fori_loop(0, BW // LANES, transpose_grp, 0)

            h0 = pl.multiple_of(g * HCH, HCH)
            pltpu.async_copy(
                trow_bufs[b],
                out_hbm.at[pl.ds(h0, HCH), pl.ds(0, EMB_DIM), pl.ds(b0, BW)],
                s_o[b])

            @pl.when(g + NBUF < NCHUNK)
            def _():
                start_idx(g + NBUF, b)
        return carry

    lax.fori_loop(0, NCHUNK // NBUF, outer, 0)
    wait_out(0)
    wait_out(1)


def kernel(idx, emb):
    idx_t = jnp.swapaxes(idx.astype(jnp.int32), 0, 1)
    out_t = _sc_gather(idx_t, emb)
    return jnp.transpose(out_t, (2, 0, 1))


# speedup vs baseline: 5.0962x; 1.3876x over previous
import functools

import jax
import jax.numpy as jnp
from jax import lax
from jax.experimental import pallas as pl
from jax.experimental.pallas import tpu as pltpu
from jax.experimental.pallas import tpu_sc as plsc

NUM_CLS = 1000000
EMB_DIM = 16
BATCH = 16384
HIST = 200

NC = 2
NS = 16
NW = NC * NS
BW = BATCH // NW
HCH = 2
NCHUNK = HIST // HCH
NBUF = 2
LANES = 16

_mesh = plsc.VectorSubcoreMesh(core_axis_name="c", subcore_axis_name="s")


@functools.partial(
    pl.kernel,
    out_type=jax.ShapeDtypeStruct((HIST, EMB_DIM, BATCH), jnp.float32),
    mesh=_mesh,
    scratch_types=[
        pltpu.VMEM((HCH, BW), jnp.int32),
        pltpu.VMEM((HCH, BW), jnp.int32),
        pltpu.VMEM((HCH * BW, EMB_DIM), jnp.float32),
        pltpu.VMEM((HCH * BW, EMB_DIM), jnp.float32),
        pltpu.VMEM((HCH, EMB_DIM, BW), jnp.float32),
        pltpu.VMEM((HCH, EMB_DIM, BW), jnp.float32),
        pltpu.SemaphoreType.DMA,
        pltpu.SemaphoreType.DMA,
        pltpu.SemaphoreType.DMA,
        pltpu.SemaphoreType.DMA,
        pltpu.SemaphoreType.DMA,
        pltpu.SemaphoreType.DMA,
    ],
    compiler_params=pltpu.CompilerParams(
        use_tc_tiling_on_sc=False, needs_layout_passes=False),
)
def _sc_gather(idx_hbm, emb_hbm, out_hbm,
               idx_v0, idx_v1, rows_v0, rows_v1, trows_v0, trows_v1,
               s_i0, s_i1, s_g0, s_g1, s_o0, s_o1):
    wid = lax.axis_index("s") * NC + lax.axis_index("c")
    b0 = pl.multiple_of(wid * BW, BW)
    idx_bufs = (idx_v0, idx_v1)
    row_bufs = (rows_v0, rows_v1)
    trow_bufs = (trows_v0, trows_v1)
    s_i = (s_i0, s_i1)
    s_g = (s_g0, s_g1)
    s_o = (s_o0, s_o1)

    lane_iota = lax.iota(jnp.int32, LANES)
    e_splats = [jnp.full((LANES,), e, jnp.int32) for e in range(EMB_DIM)]

    def start_idx(g, b):
        h0 = pl.multiple_of(g * HCH, HCH)
        pltpu.async_copy(
            idx_hbm.at[pl.ds(h0, HCH), pl.ds(b0, BW)], idx_bufs[b], s_i[b])

    def wait_idx(b):
        pltpu.make_async_copy(
            idx_hbm.at[pl.ds(0, HCH), pl.ds(0, BW)], idx_bufs[b], s_i[b]).wait()

    def wait_out(b):
        pltpu.make_async_copy(
            trow_bufs[b],
            out_hbm.at[pl.ds(0, HCH), pl.ds(0, EMB_DIM), pl.ds(0, BW)],
            s_o[b]).wait()

    start_idx(0, 0)
    start_idx(1, 1)

    def outer(t, carry):
        for b in range(NBUF):
            g = t * NBUF + b
            wait_idx(b)

            @pl.when(t > 0)
            def _():
                wait_out(b)

            copies = [
                pltpu.async_copy(
                    emb_hbm.at[idx_bufs[b].at[hh]],
                    row_bufs[b].at[pl.ds(hh * BW, BW)],
                    s_g[b],
                )
                for hh in range(HCH)
            ]
            for c in copies:
                c.wait()

            @plsc.parallel_loop(0, BW // LANES, 1, unroll=2)
            def _(i):
                base = pl.multiple_of(i * LANES, LANES)
                for hh in range(HCH):
                    row_idx = hh * BW + base + lane_iota
                    for e in range(EMB_DIM):
                        vec = plsc.load_gather(
                            row_bufs[b], [row_idx, e_splats[e]])
                        trow_bufs[b][hh, e, pl.ds(base, LANES)] = vec

            h0 = pl.multiple_of(g * HCH, HCH)
            pltpu.async_copy(
                trow_bufs[b],
                out_hbm.at[pl.ds(h0, HCH), pl.ds(0, EMB_DIM), pl.ds(b0, BW)],
                s_o[b])

            @pl.when(g + NBUF < NCHUNK)
            def _():
                start_idx(g + NBUF, b)
        return carry

    lax.fori_loop(0, NCHUNK // NBUF, outer, 0)
    wait_out(0)
    wait_out(1)


def kernel(idx, emb):
    idx_t = jnp.swapaxes(idx.astype(jnp.int32), 0, 1)
    out_t = _sc_gather(idx_t, emb)
    return jnp.transpose(out_t, (2, 0, 1))


# speedup vs baseline: 6.0590x; 1.1889x over previous
import functools

import jax
import jax.numpy as jnp
from jax import lax
from jax.experimental import pallas as pl
from jax.experimental.pallas import tpu as pltpu
from jax.experimental.pallas import tpu_sc as plsc

NUM_CLS = 1000000
EMB_DIM = 16
BATCH = 16384
HIST = 200

NC = 2
NS = 16
NW = NC * NS
BW = BATCH // NW
HCH = 2
NCHUNK = HIST // HCH
NBUF = 2
LANES = 16

_mesh = plsc.VectorSubcoreMesh(core_axis_name="c", subcore_axis_name="s")


@functools.partial(
    pl.kernel,
    out_type=jax.ShapeDtypeStruct(
        (HIST, EMB_DIM // 8, BATCH // 128, 8, 128), jnp.float32),
    mesh=_mesh,
    scratch_types=[
        pltpu.VMEM((HCH, BW), jnp.int32),
        pltpu.VMEM((HCH, BW), jnp.int32),
        pltpu.VMEM((HCH * BW, EMB_DIM), jnp.float32),
        pltpu.VMEM((HCH * BW, EMB_DIM), jnp.float32),
        pltpu.VMEM((HCH, EMB_DIM // 8, BW // 128, 8, 128), jnp.float32),
        pltpu.VMEM((HCH, EMB_DIM // 8, BW // 128, 8, 128), jnp.float32),
        pltpu.SemaphoreType.DMA,
        pltpu.SemaphoreType.DMA,
        pltpu.SemaphoreType.DMA,
        pltpu.SemaphoreType.DMA,
        pltpu.SemaphoreType.DMA,
        pltpu.SemaphoreType.DMA,
    ],
    compiler_params=pltpu.CompilerParams(
        use_tc_tiling_on_sc=False, needs_layout_passes=False),
)
def _sc_gather(idx_hbm, emb_hbm, out_hbm,
               idx_v0, idx_v1, rows_v0, rows_v1, trows_v0, trows_v1,
               s_i0, s_i1, s_g0, s_g1, s_o0, s_o1):
    wid = lax.axis_index("s") * NC + lax.axis_index("c")
    b0 = pl.multiple_of(wid * BW, BW)
    idx_bufs = (idx_v0, idx_v1)
    row_bufs = (rows_v0, rows_v1)
    trow_bufs = (trows_v0, trows_v1)
    s_i = (s_i0, s_i1)
    s_g = (s_g0, s_g1)
    s_o = (s_o0, s_o1)

    lane_iota = lax.iota(jnp.int32, LANES)
    e_splats = [jnp.full((LANES,), e, jnp.int32) for e in range(EMB_DIM)]

    def start_idx(g, b):
        h0 = pl.multiple_of(g * HCH, HCH)
        pltpu.async_copy(
            idx_hbm.at[pl.ds(h0, HCH), pl.ds(b0, BW)], idx_bufs[b], s_i[b])

    def wait_idx(b):
        pltpu.make_async_copy(
            idx_hbm.at[pl.ds(0, HCH), pl.ds(0, BW)], idx_bufs[b], s_i[b]).wait()

    def wait_out(b):
        pltpu.make_async_copy(
            trow_bufs[b],
            out_hbm.at[pl.ds(0, HCH), :, pl.ds(0, BW // 128)],
            s_o[b]).wait()

    start_idx(0, 0)
    start_idx(1, 1)

    def outer(t, carry):
        for b in range(NBUF):
            g = t * NBUF + b
            wait_idx(b)

            @pl.when(t > 0)
            def _():
                wait_out(b)

            copies = [
                pltpu.async_copy(
                    emb_hbm.at[idx_bufs[b].at[hh]],
                    row_bufs[b].at[pl.ds(hh * BW, BW)],
                    s_g[b],
                )
                for hh in range(HCH)
            ]
            for c in copies:
                c.wait()

            @plsc.parallel_loop(0, BW // LANES, 1, unroll=2)
            def _(i):
                base = pl.multiple_of(i * LANES, LANES)
                bt = i // 8
                bg = pl.multiple_of((i % 8) * LANES, LANES)
                for hh in range(HCH):
                    row_idx = hh * BW + base + lane_iota
                    for e in range(EMB_DIM):
                        vec = plsc.load_gather(
                            row_bufs[b], [row_idx, e_splats[e]])
                        trow_bufs[b][hh, e // 8, bt, e % 8,
                                     pl.ds(bg, LANES)] = vec

            h0 = pl.multiple_of(g * HCH, HCH)
            pltpu.async_copy(
                trow_bufs[b],
                out_hbm.at[pl.ds(h0, HCH), :,
                           pl.ds(wid * (BW // 128), BW // 128)],
                s_o[b])

            @pl.when(g + NBUF < NCHUNK)
            def _():
                start_idx(g + NBUF, b)
        return carry

    lax.fori_loop(0, NCHUNK // NBUF, outer, 0)
    wait_out(0)
    wait_out(1)


def kernel(idx, emb):
    idx_t = jnp.swapaxes(idx.astype(jnp.int32), 0, 1)
    out5 = _sc_gather(idx_t, emb)
    return jnp.transpose(out5, (2, 4, 0, 1, 3)).reshape(BATCH, HIST, EMB_DIM)


# speedup vs baseline: 6.3898x; 1.0546x over previous
import functools

import jax
import jax.numpy as jnp
from jax import lax
from jax.experimental import pallas as pl
from jax.experimental.pallas import tpu as pltpu
from jax.experimental.pallas import tpu_sc as plsc

NUM_CLS = 1000000
EMB_DIM = 16
BATCH = 16384
HIST = 200

NC = 2
NS = 16
NW = NC * NS
BW = BATCH // NW
HCH = 2
NCHUNK = HIST // HCH
NBUF = 2
LANES = 16

_mesh = plsc.VectorSubcoreMesh(core_axis_name="c", subcore_axis_name="s")


@functools.partial(
    pl.kernel,
    out_type=jax.ShapeDtypeStruct(
        (HIST, EMB_DIM // 8, BATCH // 128, 8, 128), jnp.float32),
    mesh=_mesh,
    scratch_types=[
        pltpu.VMEM((HCH, BW), jnp.int32),
        pltpu.VMEM((HCH, BW), jnp.int32),
        pltpu.VMEM((HCH * BW, EMB_DIM), jnp.float32),
        pltpu.VMEM((HCH * BW, EMB_DIM), jnp.float32),
        pltpu.VMEM((HCH, EMB_DIM // 8, BW // 128, 8, 128), jnp.float32),
        pltpu.VMEM((HCH, EMB_DIM // 8, BW // 128, 8, 128), jnp.float32),
        pltpu.SemaphoreType.DMA,
        pltpu.SemaphoreType.DMA,
        pltpu.SemaphoreType.DMA,
        pltpu.SemaphoreType.DMA,
        pltpu.SemaphoreType.DMA,
        pltpu.SemaphoreType.DMA,
    ],
    compiler_params=pltpu.CompilerParams(
        use_tc_tiling_on_sc=False, needs_layout_passes=False),
)
def _sc_gather(idx_hbm, emb_hbm, out_hbm,
               idx_v0, idx_v1, rows_v0, rows_v1, trows_v0, trows_v1,
               s_i0, s_i1, s_g0, s_g1, s_o0, s_o1):
    wid = lax.axis_index("s") * NC + lax.axis_index("c")
    b0 = pl.multiple_of(wid * BW, BW)
    idx_bufs = (idx_v0, idx_v1)
    row_bufs = (rows_v0, rows_v1)
    trow_bufs = (trows_v0, trows_v1)
    s_i = (s_i0, s_i1)
    s_g = (s_g0, s_g1)
    s_o = (s_o0, s_o1)

    lane_iota = lax.iota(jnp.int32, LANES)
    e_splats = [jnp.full((LANES,), e, jnp.int32) for e in range(EMB_DIM)]

    def start_idx(g, b):
        h0 = pl.multiple_of(g * HCH, HCH)
        pltpu.async_copy(
            idx_hbm.at[pl.ds(h0, HCH), pl.ds(b0, BW)], idx_bufs[b], s_i[b])

    def wait_idx(b):
        pltpu.make_async_copy(
            idx_hbm.at[pl.ds(0, HCH), pl.ds(0, BW)], idx_bufs[b], s_i[b]).wait()

    def wait_out(b):
        pltpu.make_async_copy(
            trow_bufs[b],
            out_hbm.at[pl.ds(0, HCH), :, pl.ds(0, BW // 128)],
            s_o[b]).wait()

    def fire_gathers(b):
        for hh in range(HCH):
            pltpu.async_copy(
                emb_hbm.at[idx_bufs[b].at[hh]],
                row_bufs[b].at[pl.ds(hh * BW, BW)],
                s_g[b],
            )

    def drain_gathers(b):
        for hh in range(HCH):
            pltpu.make_async_copy(
                emb_hbm.at[idx_bufs[b].at[hh]],
                row_bufs[b].at[pl.ds(hh * BW, BW)],
                s_g[b],
            ).wait()

    def transpose(b):
        @plsc.parallel_loop(0, BW // LANES, 1, unroll=4)
        def _(i):
            base = pl.multiple_of(i * LANES, LANES)
            bt = i // 8
            bg = pl.multiple_of((i % 8) * LANES, LANES)
            for hh in range(HCH):
                row_idx = hh * BW + base + lane_iota
                for e in range(EMB_DIM):
                    vec = plsc.load_gather(
                        row_bufs[b], [row_idx, e_splats[e]])
                    trow_bufs[b][hh, e // 8, bt, e % 8,
                                 pl.ds(bg, LANES)] = vec

    def start_out(g, b):
        h0 = pl.multiple_of(g * HCH, HCH)
        pltpu.async_copy(
            trow_bufs[b],
            out_hbm.at[pl.ds(h0, HCH), :,
                       pl.ds(wid * (BW // 128), BW // 128)],
            s_o[b])

    start_idx(0, 0)
    start_idx(1, 1)
    wait_idx(0)
    fire_gathers(0)

    def outer(t, carry):
        for b in range(NBUF):
            g = t * NBUF + b
            bn = 1 - b

            @pl.when(g + 1 < NCHUNK)
            def _():
                wait_idx(bn)
                fire_gathers(bn)

            drain_gathers(b)

            @pl.when(t > 0)
            def _():
                wait_out(b)

            transpose(b)
            start_out(g, b)

            @pl.when(g + NBUF < NCHUNK)
            def _():
                start_idx(g + NBUF, b)
        return carry

    lax.fori_loop(0, NCHUNK // NBUF, outer, 0)
    wait_out(0)
    wait_out(1)


def kernel(idx, emb):
    idx_t = jnp.swapaxes(idx.astype(jnp.int32), 0, 1)
    out5 = _sc_gather(idx_t, emb)
    return jnp.transpose(out5, (2, 4, 0, 1, 3)).reshape(BATCH, HIST, EMB_DIM)
